# baseline (speedup 1.0000x reference)
import functools

import jax
import jax.numpy as jnp
from jax import lax
from jax.experimental import pallas as pl
from jax.experimental.pallas import tpu as pltpu
from jax.experimental.pallas import tpu_sc as plsc

N_NODES = 10000
N_EDGES = 320000
DIN = 128
DOUT = 64
NHEADS = 4
NCLS = 16
SLOPE = 0.1

NC = 2
NS = 16
NW = NC * NS
EPW = N_EDGES // NW
CH = 80
NG16 = CH // 16
NCHUNK = EPW // CH
RPT = 632
NPAD = RPT * NS
PAY = 128
WSTRIDE = NHEADS * CH
BCH1 = 25
NB1 = NCHUNK // BCH1
BCH2 = 5
NB2 = NCHUNK // BCH2

_MESH = plsc.VectorSubcoreMesh(
    core_axis_name="c", subcore_axis_name="s", num_cores=NC, num_subcores=NS
)
_SC_PARAMS = pltpu.CompilerParams(needs_layout_passes=False)

_ZCHUNKS = ((0, 80), (80, 80), (160, 80), (240, 80), (320, 80),
            (400, 80), (480, 80), (560, 72))


_BA = 632


def _mm1_body(x1_ref, x2_ref, ws_ref, wt_ref, axs_ref, axt_ref,
              hs_ref, ht_ref, eset_ref):
    x1 = x1_ref[...]
    x2 = x2_ref[...]
    hs_ref[0] = jnp.dot(x1, ws_ref[0], preferred_element_type=jnp.float32)
    ht_ref[0] = jnp.dot(x2, wt_ref[0], preferred_element_type=jnp.float32)
    eset_ref[...] = (
        jnp.dot(x1, axs_ref[...], preferred_element_type=jnp.float32)
        + jnp.dot(x2, axt_ref[...], preferred_element_type=jnp.float32)
    )


_mm1 = pl.pallas_call(
    _mm1_body,
    grid=(NPAD // _BA, 2),
    in_specs=[
        pl.BlockSpec((_BA, DIN), lambda nb, hp: (nb, 0)),
        pl.BlockSpec((_BA, DIN), lambda nb, hp: (nb, 0)),
        pl.BlockSpec((1, DIN, PAY), lambda nb, hp: (hp, 0, 0)),
        pl.BlockSpec((1, DIN, PAY), lambda nb, hp: (hp, 0, 0)),
        pl.BlockSpec((DIN, 8), lambda nb, hp: (0, 0)),
        pl.BlockSpec((DIN, 8), lambda nb, hp: (0, 0)),
    ],
    out_specs=[
        pl.BlockSpec((1, _BA, PAY), lambda nb, hp: (hp, nb, 0)),
        pl.BlockSpec((1, _BA, PAY), lambda nb, hp: (hp, nb, 0)),
        pl.BlockSpec((_BA, 8), lambda nb, hp: (nb, 0)),
    ],
    out_shape=[
        jax.ShapeDtypeStruct((2, NPAD, PAY), jnp.float32),
        jax.ShapeDtypeStruct((2, NPAD, PAY), jnp.float32),
        jax.ShapeDtypeStruct((NPAD, 8), jnp.float32),
    ],
)


@functools.partial(
    pl.kernel,
    out_type=jax.ShapeDtypeStruct((N_EDGES * NHEADS,), jnp.float32),
    mesh=_MESH,
    compiler_params=_SC_PARAMS,
    scratch_types=[
        pltpu.VMEM((NPAD * 8,), jnp.float32),
        pltpu.VMEM((BCH1 * CH,), jnp.int32),
        pltpu.VMEM((BCH1 * CH,), jnp.int32),
        pltpu.VMEM((BCH1 * WSTRIDE,), jnp.float32),
    ],
)
def _edge_logits(ei_hbm, ej_hbm, eset_hbm, w_hbm, eset_v, ib, jb, wt):
    wid = lax.axis_index("s") * NC + lax.axis_index("c")
    pltpu.sync_copy(eset_hbm, eset_v)

    def batch(b, carry):
        base = wid * EPW + b * BCH1 * CH
        pltpu.sync_copy(ei_hbm.at[pl.ds(base, BCH1 * CH)], ib)
        pltpu.sync_copy(ej_hbm.at[pl.ds(base, BCH1 * CH)], jb)

        def step(c, c2):
            for kk in range(NG16):
                coff = c * CH + kk * 16
                ii8 = ib[pl.ds(coff, 16)] * 8
                jj8 = jb[pl.ds(coff, 16)] * 8
                for h in range(NHEADS):
                    es = plsc.load_gather(
                        eset_v, [jj8 + jnp.full((16,), h, jnp.int32)])
                    et = plsc.load_gather(
                        eset_v, [ii8 + jnp.full((16,), NHEADS + h, jnp.int32)])
                    x = es + et
                    x = jnp.where(x >= 0, x, SLOPE * x)
                    wt[pl.ds(c * WSTRIDE + h * CH + kk * 16, 16)] = jnp.exp(x)
            return c2

        lax.fori_loop(0, BCH1, step, 0)
        pltpu.sync_copy(wt, w_hbm.at[pl.ds((wid * NCHUNK + b * BCH1) *
                                           WSTRIDE, BCH1 * WSTRIDE)])
        return carry

    lax.fori_loop(0, NB1, batch, 0)


@functools.partial(
    pl.kernel,
    out_type=jax.ShapeDtypeStruct((NW * NPAD * 8,), jnp.float32),
    mesh=_MESH,
    compiler_params=_SC_PARAMS,
    scratch_types=[
        pltpu.VMEM((NPAD * 8,), jnp.float32),
        pltpu.VMEM((BCH1 * CH,), jnp.int32),
        pltpu.VMEM((BCH1 * CH,), jnp.int32),
        pltpu.VMEM((BCH1 * WSTRIDE,), jnp.float32),
    ],
)
def _sums(ei_hbm, ej_hbm, w_hbm, out_hbm, s_priv, ib, jb, wt):
    wid = lax.axis_index("s") * NC + lax.axis_index("c")
    z16 = jnp.zeros((16,), jnp.float32)

    def zero(r, carry):
        s_priv[pl.ds(r * 16, 16)] = z16
        return carry

    lax.fori_loop(0, NPAD * 8 // 16, zero, 0)

    def batch(b, carry):
        base = wid * EPW + b * BCH1 * CH
        pltpu.sync_copy(ei_hbm.at[pl.ds(base, BCH1 * CH)], ib)
        pltpu.sync_copy(ej_hbm.at[pl.ds(base, BCH1 * CH)], jb)
        pltpu.sync_copy(w_hbm.at[pl.ds((wid * NCHUNK + b * BCH1) * WSTRIDE,
                                       BCH1 * WSTRIDE)], wt)

        def step(c, c2):
            for kk in range(NG16):
                coff = c * CH + kk * 16
                ii8 = ib[pl.ds(coff, 16)] * 8
                jj8 = jb[pl.ds(coff, 16)] * 8
                for h in range(NHEADS):
                    wv = wt[pl.ds(c * WSTRIDE + h * CH + kk * 16, 16)]
                    plsc.addupdate_scatter(
                        s_priv, [ii8 + jnp.full((16,), h, jnp.int32)], wv)
                    plsc.addupdate_scatter(
                        s_priv, [jj8 + jnp.full((16,), 4 + h, jnp.int32)], wv)
            return c2

        lax.fori_loop(0, BCH1, step, 0)
        return carry

    lax.fori_loop(0, NB1, batch, 0)
    pltpu.sync_copy(s_priv, out_hbm.at[pl.ds(wid * (NPAD * 8), NPAD * 8)])


def _fill16(dst, src, off):
    for gg in range(NG16):
        dst[pl.ds(gg * 16, 16)] = src[pl.ds(off + gg * 16, 16)]


def _make_scatter1(pp, h0):
    @functools.partial(
        pl.kernel,
        out_type=jax.ShapeDtypeStruct((NC, NPAD, PAY), jnp.float32),
        mesh=_MESH,
        compiler_params=_SC_PARAMS,
        scratch_types=[
            pltpu.VMEM_SHARED((NPAD, PAY), jnp.float32),
            pltpu.VMEM((BCH1 * CH,), jnp.int32),
            pltpu.VMEM((BCH1 * CH,), jnp.int32),
            pltpu.VMEM((BCH1 * WSTRIDE,), jnp.float32),
            pltpu.VMEM((CH,), jnp.int32),
            pltpu.VMEM((CH,), jnp.int32),
            pltpu.VMEM((CH,), jnp.int32),
            pltpu.VMEM((CH,), jnp.int32),
            pltpu.VMEM((CH, PAY), jnp.float32),
            pltpu.VMEM((CH, PAY), jnp.float32),
            pltpu.VMEM((CH, PAY), jnp.float32),
            pltpu.SemaphoreType.DMA,
            pltpu.SemaphoreType.DMA,
            pltpu.SemaphoreType.DMA,
        ],
    )
    def _scat(keys_hbm, sidx_hbm, w_hbm, src3_hbm, out_hbm,
              acc, kbig, sbig, wbig, kb, sb0, sb1, sb2, rows0, rows1, rows2,
              sem0, sem1, sem2):
        cid = lax.axis_index("c")
        sid = lax.axis_index("s")
        wid = sid * NC + cid
        z16 = jnp.zeros((16,), jnp.float32)
        r0 = sid * RPT
        src_hbm = src3_hbm.at[pp]

        def zero_acc():
            def zrow(r, carry):
                for cc in range(PAY // 16):
                    rows0[r, pl.ds(cc * 16, 16)] = z16
                return carry

            lax.fori_loop(0, CH, zrow, 0)
            for off, nr in _ZCHUNKS:
                pltpu.sync_copy(rows0.at[pl.ds(0, nr)],
                                acc.at[pl.ds(r0 + off, nr)])

        def scale(rows, woff):
            def body(gg, c2):
                wv0 = wbig[pl.ds(woff + gg * 16, 16)]
                wv1 = wbig[pl.ds(woff + CH + gg * 16, 16)]
                for kp in range(16):
                    k = gg * 16 + kp
                    w0 = wv0[kp]
                    w1 = wv1[kp]
                    for cc in range(4):
                        rows[k, pl.ds(cc * 16, 16)] = (
                            rows[k, pl.ds(cc * 16, 16)] * w0)
                    for cc in range(4, 8):
                        rows[k, pl.ds(cc * 16, 16)] = (
                            rows[k, pl.ds(cc * 16, 16)] * w1)
                return c2

            lax.fori_loop(0, NG16, body, 0)

        zero_acc()
        plsc.subcore_barrier()

        ring = ((sb0, rows0, sem0), (sb1, rows1, sem1), (sb2, rows2, sem2))

        def proc(c, slot):
            sbx, rowsx, semx = ring[slot]
            _fill16(kb, kbig, c * CH)
            pltpu.make_async_copy(src_hbm.at[sbx], rowsx, semx).wait()
            scale(rowsx, c * WSTRIDE + h0 * CH)
            pltpu.sync_copy(rowsx, acc.at[kb], add=True)

        def issue(c, slot):
            sbx, rowsx, semx = ring[slot]
            _fill16(sbx, sbig, c * CH)
            pltpu.async_copy(src_hbm.at[sbx], rowsx, semx)

        def batch(b, carry):
            ib = wid * EPW + b * BCH1 * CH
            wb0 = (wid * NCHUNK + b * BCH1) * WSTRIDE
            pltpu.sync_copy(keys_hbm.at[pl.ds(ib, BCH1 * CH)], kbig)
            pltpu.sync_copy(sidx_hbm.at[pl.ds(ib, BCH1 * CH)], sbig)
            pltpu.sync_copy(w_hbm.at[pl.ds(wb0, BCH1 * WSTRIDE)], wbig)

            issue(0, 0)
            issue(1, 1)

            def triple(t, c2):
                c0 = 3 * t
                issue(c0 + 2, 2)
                proc(c0, 0)
                issue(c0 + 3, 0)
                proc(c0 + 1, 1)
                issue(c0 + 4, 1)
                proc(c0 + 2, 2)
                return c2

            nt = (BCH1 - 4) // 3
            lax.fori_loop(0, nt, triple, 0)
            c = 3 * nt
            issue(c + 2, (c + 2) % 3)
            proc(c, c % 3)
            issue(c + 3, (c + 3) % 3)
            proc(c + 1, (c + 1) % 3)
            proc(c + 2, (c + 2) % 3)
            proc(c + 3, (c + 3) % 3)
            return carry

        lax.fori_loop(0, NB1, batch, 0)
        plsc.subcore_barrier()
        pltpu.sync_copy(acc.at[pl.ds(r0, RPT)],
                        out_hbm.at[cid, pl.ds(r0, RPT)])

    return _scat


_scatter1_p0 = _make_scatter1(0, 0)
_scatter1_p1 = _make_scatter1(1, 2)


_BD = 632


def _elu(x):
    return jnp.where(x > 0, x, jnp.exp(jnp.minimum(x, 0.0)) - 1.0)


def _mid_body(acc_ref, sv_ref, wso_ref, wto_ref,
              a2s_ref, a2t_ref, hs2_ref, ht2_ref, eset2_ref):
    s = jnp.sum(sv_ref[...], axis=0)

    def build(p0, scol):
        cols = []
        for idx in range(2):
            a = acc_ref[p0 + 2 * idx] + acc_ref[p0 + 2 * idx + 1]
            s0 = s[:, scol + 2 * idx:scol + 2 * idx + 1]
            s1 = s[:, scol + 2 * idx + 1:scol + 2 * idx + 2]
            cols.append(a[:, 0:DOUT] / (s0 + 1e-16))
            cols.append(a[:, DOUT:PAY] / (s1 + 1e-16))
        return _elu(jnp.concatenate(cols, axis=1))

    hts = build(0, 0)
    hst = build(4, 4)
    hs2 = jnp.dot(hts, wso_ref[...], preferred_element_type=jnp.float32)
    ht2 = jnp.dot(hst, wto_ref[...], preferred_element_type=jnp.float32)
    zpad = jnp.zeros((hs2.shape[0], PAY - NCLS), jnp.float32)
    hs2_ref[...] = jnp.concatenate([hs2, zpad], axis=1)
    ht2_ref[...] = jnp.concatenate([ht2, zpad], axis=1)
    eset2_ref[...] = (
        jnp.dot(hs2, a2s_ref[...], preferred_element_type=jnp.float32)
        + jnp.dot(ht2, a2t_ref[...], preferred_element_type=jnp.float32)
    )


_mid = pl.pallas_call(
    _mid_body,
    grid=(NPAD // _BD,),
    in_specs=[
        pl.BlockSpec((4 * NC, _BD, PAY), lambda nb: (0, nb, 0)),
        pl.BlockSpec((NW, _BD, 8), lambda nb: (0, nb, 0)),
        pl.BlockSpec((NHEADS * DOUT, NCLS), lambda nb: (0, 0)),
        pl.BlockSpec((NHEADS * DOUT, NCLS), lambda nb: (0, 0)),
        pl.BlockSpec((NCLS, 8), lambda nb: (0, 0)),
        pl.BlockSpec((NCLS, 8), lambda nb: (0, 0)),
    ],
    out_specs=[
        pl.BlockSpec((_BD, PAY), lambda nb: (nb, 0)),
        pl.BlockSpec((_BD, PAY), lambda nb: (nb, 0)),
        pl.BlockSpec((_BD, 8), lambda nb: (nb, 0)),
    ],
    out_shape=[
        jax.ShapeDtypeStruct((NPAD, PAY), jnp.float32),
        jax.ShapeDtypeStruct((NPAD, PAY), jnp.float32),
        jax.ShapeDtypeStruct((NPAD, 8), jnp.float32),
    ],
)


@functools.partial(
    pl.kernel,
    out_type=jax.ShapeDtypeStruct((2, NC, NPAD, PAY), jnp.float32),
    mesh=_MESH,
    compiler_params=_SC_PARAMS,
    scratch_types=[
        pltpu.VMEM_SHARED((NPAD, PAY), jnp.float32),
        pltpu.VMEM((NPAD * 2,), jnp.float32),
        pltpu.VMEM((BCH2 * CH,), jnp.int32),
        pltpu.VMEM((BCH2 * CH,), jnp.int32),
        pltpu.VMEM((CH,), jnp.int32),
        pltpu.VMEM((CH,), jnp.int32),
        pltpu.VMEM((CH,), jnp.int32),
        pltpu.VMEM((CH,), jnp.float32),
        pltpu.VMEM((CH, PAY), jnp.float32),
        pltpu.VMEM((CH, PAY), jnp.float32),
        pltpu.SemaphoreType.DMA,
        pltpu.SemaphoreType.DMA,
    ],
)
def _scatter2(ei_hbm, ej_hbm, hs2_hbm, ht2_hbm, eset2_hbm, out_hbm,
              acc, eset_v, kbig, sbig, kb, sb0, sb1, wbuf,
              rows0, rows1, sem0, sem1):
    cid = lax.axis_index("c")
    sid = lax.axis_index("s")
    wid = sid * NC + cid
    pltpu.sync_copy(eset2_hbm, eset_v)
    z16 = jnp.zeros((16,), jnp.float32)
    r0 = sid * RPT
    lane = lax.iota(jnp.int32, 16)

    def zero_acc():
        def zrow(r, carry):
            for cc in range(PAY // 16):
                rows0[r, pl.ds(cc * 16, 16)] = z16
            return carry

        lax.fori_loop(0, CH, zrow, 0)
        for off, nr in _ZCHUNKS:
            pltpu.sync_copy(rows0.at[pl.ds(0, nr)],
                            acc.at[pl.ds(r0 + off, nr)])

    def scale(rows):
        def body(gg, c2):
            wv = wbuf[pl.ds(gg * 16, 16)]
            for kp in range(16):
                k = gg * 16 + kp
                w = wv[kp]
                rows[k, pl.ds(0, 16)] = rows[k, pl.ds(0, 16)] * w
                rows[k, pl.ds(16, 16)] = jnp.where(lane == 0, w, 0.0)
            return c2

        lax.fori_loop(0, NG16, body, 0)

    for p, (keys_hbm, sidx_hbm, src_hbm, es_on_sidx) in enumerate((
            (ei_hbm, ej_hbm, hs2_hbm, True),
            (ej_hbm, ei_hbm, ht2_hbm, False))):

        def weights(coff):
            for kk in range(NG16):
                kk8 = kbig[pl.ds(coff + kk * 16, 16)] * 2
                ss8 = sbig[pl.ds(coff + kk * 16, 16)] * 2
                if es_on_sidx:
                    es = plsc.load_gather(eset_v, [ss8])
                    et = plsc.load_gather(
                        eset_v, [kk8 + jnp.ones((16,), jnp.int32)])
                else:
                    es = plsc.load_gather(eset_v, [kk8])
                    et = plsc.load_gather(
                        eset_v, [ss8 + jnp.ones((16,), jnp.int32)])
                x = es + et
                x = jnp.where(x >= 0, x, SLOPE * x)
                wbuf[pl.ds(kk * 16, 16)] = jnp.exp(x)

        zero_acc()
        plsc.subcore_barrier()

        def batch(b, carry):
            ib = wid * EPW + b * BCH2 * CH
            pltpu.sync_copy(keys_hbm.at[pl.ds(ib, BCH2 * CH)], kbig)
            pltpu.sync_copy(sidx_hbm.at[pl.ds(ib, BCH2 * CH)], sbig)

            _fill16(sb0, sbig, 0)
            pltpu.async_copy(src_hbm.at[sb0], rows0, sem0)

            def pair(t, c2):
                c0 = 2 * t
                _fill16(sb1, sbig, (c0 + 1) * CH)
                pltpu.async_copy(src_hbm.at[sb1], rows1, sem1)
                _fill16(kb, kbig, c0 * CH)
                weights(c0 * CH)
                pltpu.make_async_copy(src_hbm.at[sb0], rows0, sem0).wait()
                scale(rows0)
                pltpu.sync_copy(rows0, acc.at[kb], add=True)
                _fill16(sb0, sbig, (c0 + 2) * CH)
                pltpu.async_copy(src_hbm.at[sb0], rows0, sem0)
                _fill16(kb, kbig, (c0 + 1) * CH)
                weights((c0 + 1) * CH)
                pltpu.make_async_copy(src_hbm.at[sb1], rows1, sem1).wait()
                scale(rows1)
                pltpu.sync_copy(rows1, acc.at[kb], add=True)
                return c2

            lax.fori_loop(0, (BCH2 - 1) // 2, pair, 0)
            _fill16(kb, kbig, (BCH2 - 1) * CH)
            weights((BCH2 - 1) * CH)
            pltpu.make_async_copy(src_hbm.at[sb0], rows0, sem0).wait()
            scale(rows0)
            pltpu.sync_copy(rows0, acc.at[kb], add=True)
            return carry

        lax.fori_loop(0, NB2, batch, 0)
        plsc.subcore_barrier()
        pltpu.sync_copy(acc.at[pl.ds(r0, RPT)],
                        out_hbm.at[p, cid, pl.ds(r0, RPT)])


_BG = 632


def _fin_body(a2_ref, ost_ref, ots_ref):
    def fin(p0):
        a = a2_ref[p0] + a2_ref[p0 + 1]
        s = a[:, NCLS:NCLS + 1]
        v = _elu(a[:, 0:NCLS] / (s + 1e-16))
        m = jnp.max(v, axis=1, keepdims=True)
        lse = jnp.log(jnp.sum(jnp.exp(v - m), axis=1, keepdims=True)) + m
        return v - lse

    ost_ref[...] = fin(2)
    ots_ref[...] = fin(0)


_fin = pl.pallas_call(
    _fin_body,
    grid=(NPAD // _BG,),
    in_specs=[
        pl.BlockSpec((2 * NC, _BG, PAY), lambda nb: (0, nb, 0)),
    ],
    out_specs=[
        pl.BlockSpec((_BG, NCLS), lambda nb: (nb, 0)),
        pl.BlockSpec((_BG, NCLS), lambda nb: (nb, 0)),
    ],
    out_shape=[
        jax.ShapeDtypeStruct((NPAD, NCLS), jnp.float32),
        jax.ShapeDtypeStruct((NPAD, NCLS), jnp.float32),
    ],
)


@jax.jit
def kernel(input1, input2, edge_index, Ws_heads, Wt_heads, a_heads,
           Ws_out, Wt_out, a_out):
    edge_index = edge_index.astype(jnp.int32)
    ei = edge_index[0]
    ej = edge_index[1]

    pad = ((0, NPAD - N_NODES), (0, 0))
    x1p = jnp.pad(input1, pad)
    x2p = jnp.pad(input2, pad)

    ws2 = jnp.stack([
        jnp.concatenate([Ws_heads[0], Ws_heads[1]], axis=1),
        jnp.concatenate([Ws_heads[2], Ws_heads[3]], axis=1),
    ])
    wt2 = jnp.stack([
        jnp.concatenate([Wt_heads[0], Wt_heads[1]], axis=1),
        jnp.concatenate([Wt_heads[2], Wt_heads[3]], axis=1),
    ])
    as_vec = jnp.einsum("hdk,hk->dh", Ws_heads, a_heads[:, :DOUT])
    at_vec = jnp.einsum("hdk,hk->dh", Wt_heads, a_heads[:, DOUT:])
    aux_s = jnp.zeros((DIN, 8), jnp.float32).at[:, :NHEADS].set(as_vec)
    aux_t = jnp.zeros((DIN, 8), jnp.float32).at[:, NHEADS:].set(at_vec)
    a2s = jnp.zeros((NCLS, 8), jnp.float32).at[:, 0].set(a_out[:NCLS])
    a2t = jnp.zeros((NCLS, 8), jnp.float32).at[:, 1].set(a_out[NCLS:])

    hs2d, ht2d, eset = _mm1(x1p, x2p, ws2, wt2, aux_s, aux_t)

    w4 = _edge_logits(ei, ej, eset.reshape(-1))
    sv = _sums(ei, ej, w4).reshape(NW, NPAD, 8)

    acc1 = jnp.stack([
        _scatter1_p0(ei, ej, w4, hs2d),
        _scatter1_p1(ei, ej, w4, hs2d),
        _scatter1_p0(ej, ei, w4, ht2d),
        _scatter1_p1(ej, ei, w4, ht2d),
    ])

    hs2, ht2, eset2 = _mid(acc1.reshape(4 * NC, NPAD, PAY), sv,
                           Ws_out, Wt_out, a2s, a2t)

    eset2c = eset2[:, :2].reshape(-1)
    a2 = _scatter2(ei, ej, hs2, ht2, eset2c)

    out_st, out_ts = _fin(a2.reshape(2 * NC, NPAD, PAY))
    return out_st[:N_NODES], out_ts[:N_NODES]


# speedup vs baseline: 1.0751x; 1.0751x over previous
import functools

import jax
import jax.numpy as jnp
from jax import lax
from jax.experimental import pallas as pl
from jax.experimental.pallas import tpu as pltpu
from jax.experimental.pallas import tpu_sc as plsc

N_NODES = 10000
N_EDGES = 320000
DIN = 128
DOUT = 64
NHEADS = 4
NCLS = 16
SLOPE = 0.1

NC = 2
NS = 16
NW = NC * NS
EPW = N_EDGES // NW
CH = 80
NG16 = CH // 16
NCHUNK = EPW // CH
RPT = 632
NPAD = RPT * NS
PAY = 128
WSTRIDE = NHEADS * CH
BCH1 = 25
NB1 = NCHUNK // BCH1
BCH2 = 5
NB2 = NCHUNK // BCH2

_MESH = plsc.VectorSubcoreMesh(
    core_axis_name="c", subcore_axis_name="s", num_cores=NC, num_subcores=NS
)
_SC_PARAMS = pltpu.CompilerParams(needs_layout_passes=False)

_ZCHUNKS = ((0, 80), (80, 80), (160, 80), (240, 80), (320, 80),
            (400, 80), (480, 80), (560, 72))


_BA = 632


def _mm1_body(x1_ref, x2_ref, ws_ref, wt_ref, axs_ref, axt_ref,
              hs_ref, ht_ref, eset_ref):
    x1 = x1_ref[...]
    x2 = x2_ref[...]
    hs_ref[0] = jnp.dot(x1, ws_ref[0], preferred_element_type=jnp.float32)
    ht_ref[0] = jnp.dot(x2, wt_ref[0], preferred_element_type=jnp.float32)
    eset_ref[...] = (
        jnp.dot(x1, axs_ref[...], preferred_element_type=jnp.float32)
        + jnp.dot(x2, axt_ref[...], preferred_element_type=jnp.float32)
    )


_mm1 = pl.pallas_call(
    _mm1_body,
    grid=(NPAD // _BA, 2),
    in_specs=[
        pl.BlockSpec((_BA, DIN), lambda nb, hp: (nb, 0)),
        pl.BlockSpec((_BA, DIN), lambda nb, hp: (nb, 0)),
        pl.BlockSpec((1, DIN, PAY), lambda nb, hp: (hp, 0, 0)),
        pl.BlockSpec((1, DIN, PAY), lambda nb, hp: (hp, 0, 0)),
        pl.BlockSpec((DIN, 8), lambda nb, hp: (0, 0)),
        pl.BlockSpec((DIN, 8), lambda nb, hp: (0, 0)),
    ],
    out_specs=[
        pl.BlockSpec((1, _BA, PAY), lambda nb, hp: (hp, nb, 0)),
        pl.BlockSpec((1, _BA, PAY), lambda nb, hp: (hp, nb, 0)),
        pl.BlockSpec((_BA, 8), lambda nb, hp: (nb, 0)),
    ],
    out_shape=[
        jax.ShapeDtypeStruct((2, NPAD, PAY), jnp.float32),
        jax.ShapeDtypeStruct((2, NPAD, PAY), jnp.float32),
        jax.ShapeDtypeStruct((NPAD, 8), jnp.float32),
    ],
)


@functools.partial(
    pl.kernel,
    out_type=jax.ShapeDtypeStruct((N_EDGES * NHEADS,), jnp.float32),
    mesh=_MESH,
    compiler_params=_SC_PARAMS,
    scratch_types=[
        pltpu.VMEM((NPAD * 8,), jnp.float32),
        pltpu.VMEM((BCH1 * CH,), jnp.int32),
        pltpu.VMEM((BCH1 * CH,), jnp.int32),
        pltpu.VMEM((BCH1 * WSTRIDE,), jnp.float32),
    ],
)
def _edge_logits(ei_hbm, ej_hbm, eset_hbm, w_hbm, eset_v, ib, jb, wt):
    wid = lax.axis_index("s") * NC + lax.axis_index("c")
    pltpu.sync_copy(eset_hbm, eset_v)

    def batch(b, carry):
        base = wid * EPW + b * BCH1 * CH
        pltpu.sync_copy(ei_hbm.at[pl.ds(base, BCH1 * CH)], ib)
        pltpu.sync_copy(ej_hbm.at[pl.ds(base, BCH1 * CH)], jb)

        def step(c, c2):
            for kk in range(NG16):
                coff = c * CH + kk * 16
                ii8 = ib[pl.ds(coff, 16)] * 8
                jj8 = jb[pl.ds(coff, 16)] * 8
                for h in range(NHEADS):
                    es = plsc.load_gather(
                        eset_v, [jj8 + jnp.full((16,), h, jnp.int32)])
                    et = plsc.load_gather(
                        eset_v, [ii8 + jnp.full((16,), NHEADS + h, jnp.int32)])
                    x = es + et
                    x = jnp.where(x >= 0, x, SLOPE * x)
                    wt[pl.ds(c * WSTRIDE + h * CH + kk * 16, 16)] = jnp.exp(x)
            return c2

        lax.fori_loop(0, BCH1, step, 0)
        pltpu.sync_copy(wt, w_hbm.at[pl.ds((wid * NCHUNK + b * BCH1) *
                                           WSTRIDE, BCH1 * WSTRIDE)])
        return carry

    lax.fori_loop(0, NB1, batch, 0)


@functools.partial(
    pl.kernel,
    out_type=jax.ShapeDtypeStruct((NW * NPAD * 8,), jnp.float32),
    mesh=_MESH,
    compiler_params=_SC_PARAMS,
    scratch_types=[
        pltpu.VMEM((NPAD * 8,), jnp.float32),
        pltpu.VMEM((BCH1 * CH,), jnp.int32),
        pltpu.VMEM((BCH1 * CH,), jnp.int32),
        pltpu.VMEM((BCH1 * WSTRIDE,), jnp.float32),
    ],
)
def _sums(ei_hbm, ej_hbm, w_hbm, out_hbm, s_priv, ib, jb, wt):
    wid = lax.axis_index("s") * NC + lax.axis_index("c")
    z16 = jnp.zeros((16,), jnp.float32)

    def zero(r, carry):
        s_priv[pl.ds(r * 16, 16)] = z16
        return carry

    lax.fori_loop(0, NPAD * 8 // 16, zero, 0)

    def batch(b, carry):
        base = wid * EPW + b * BCH1 * CH
        pltpu.sync_copy(ei_hbm.at[pl.ds(base, BCH1 * CH)], ib)
        pltpu.sync_copy(ej_hbm.at[pl.ds(base, BCH1 * CH)], jb)
        pltpu.sync_copy(w_hbm.at[pl.ds((wid * NCHUNK + b * BCH1) * WSTRIDE,
                                       BCH1 * WSTRIDE)], wt)

        def step(c, c2):
            for kk in range(NG16):
                coff = c * CH + kk * 16
                ii8 = ib[pl.ds(coff, 16)] * 8
                jj8 = jb[pl.ds(coff, 16)] * 8
                for h in range(NHEADS):
                    wv = wt[pl.ds(c * WSTRIDE + h * CH + kk * 16, 16)]
                    plsc.addupdate_scatter(
                        s_priv, [ii8 + jnp.full((16,), h, jnp.int32)], wv)
                    plsc.addupdate_scatter(
                        s_priv, [jj8 + jnp.full((16,), 4 + h, jnp.int32)], wv)
            return c2

        lax.fori_loop(0, BCH1, step, 0)
        return carry

    lax.fori_loop(0, NB1, batch, 0)
    pltpu.sync_copy(s_priv, out_hbm.at[pl.ds(wid * (NPAD * 8), NPAD * 8)])


def _fill16(dst, src, off):
    for gg in range(NG16):
        dst[pl.ds(gg * 16, 16)] = src[pl.ds(off + gg * 16, 16)]


def _make_scatter1(pp, h0):
    @functools.partial(
        pl.kernel,
        out_type=jax.ShapeDtypeStruct((NC, NPAD, PAY), jnp.float32),
        mesh=_MESH,
        compiler_params=_SC_PARAMS,
        scratch_types=[
            pltpu.VMEM_SHARED((NPAD, PAY), jnp.float32),
            pltpu.VMEM((BCH1 * CH,), jnp.int32),
            pltpu.VMEM((BCH1 * CH,), jnp.int32),
            pltpu.VMEM((BCH1 * WSTRIDE,), jnp.float32),
            pltpu.VMEM((CH,), jnp.int32),
            pltpu.VMEM((CH,), jnp.int32),
            pltpu.VMEM((CH,), jnp.int32),
            pltpu.VMEM((CH,), jnp.int32),
            pltpu.VMEM((CH, PAY), jnp.float32),
            pltpu.VMEM((CH, PAY), jnp.float32),
            pltpu.VMEM((CH, PAY), jnp.float32),
            pltpu.SemaphoreType.DMA,
            pltpu.SemaphoreType.DMA,
            pltpu.SemaphoreType.DMA,
        ],
    )
    def _scat(keys_hbm, sidx_hbm, w_hbm, src3_hbm, out_hbm,
              acc, kbig, sbig, wbig, kb, sb0, sb1, sb2, rows0, rows1, rows2,
              sem0, sem1, sem2):
        cid = lax.axis_index("c")
        sid = lax.axis_index("s")
        wid = sid * NC + cid
        z16 = jnp.zeros((16,), jnp.float32)
        r0 = sid * RPT
        src_hbm = src3_hbm.at[pp]

        def zero_acc():
            def zrow(r, carry):
                for cc in range(PAY // 16):
                    rows0[r, pl.ds(cc * 16, 16)] = z16
                return carry

            lax.fori_loop(0, CH, zrow, 0)
            for off, nr in _ZCHUNKS:
                pltpu.sync_copy(rows0.at[pl.ds(0, nr)],
                                acc.at[pl.ds(r0 + off, nr)])

        def scale(rows, woff):
            def body(gg, c2):
                wv0 = wbig[pl.ds(woff + gg * 16, 16)]
                wv1 = wbig[pl.ds(woff + CH + gg * 16, 16)]
                for kp in range(16):
                    k = gg * 16 + kp
                    w0 = wv0[kp]
                    w1 = wv1[kp]
                    for cc in range(4):
                        rows[k, pl.ds(cc * 16, 16)] = (
                            rows[k, pl.ds(cc * 16, 16)] * w0)
                    for cc in range(4, 8):
                        rows[k, pl.ds(cc * 16, 16)] = (
                            rows[k, pl.ds(cc * 16, 16)] * w1)
                return c2

            lax.fori_loop(0, NG16, body, 0)

        zero_acc()
        plsc.subcore_barrier()

        ring = ((sb0, rows0, sem0), (sb1, rows1, sem1), (sb2, rows2, sem2))

        def proc(c, slot):
            sbx, rowsx, semx = ring[slot]
            _fill16(kb, kbig, c * CH)
            pltpu.make_async_copy(src_hbm.at[sbx], rowsx, semx).wait()
            scale(rowsx, c * WSTRIDE + h0 * CH)
            pltpu.sync_copy(rowsx, acc.at[kb], add=True)

        def issue(c, slot):
            sbx, rowsx, semx = ring[slot]
            _fill16(sbx, sbig, c * CH)
            pltpu.async_copy(src_hbm.at[sbx], rowsx, semx)

        def batch(b, carry):
            ib = wid * EPW + b * BCH1 * CH
            wb0 = (wid * NCHUNK + b * BCH1) * WSTRIDE
            pltpu.sync_copy(keys_hbm.at[pl.ds(ib, BCH1 * CH)], kbig)
            pltpu.sync_copy(sidx_hbm.at[pl.ds(ib, BCH1 * CH)], sbig)
            pltpu.sync_copy(w_hbm.at[pl.ds(wb0, BCH1 * WSTRIDE)], wbig)

            issue(0, 0)
            issue(1, 1)

            def triple(t, c2):
                c0 = 3 * t
                issue(c0 + 2, 2)
                proc(c0, 0)
                issue(c0 + 3, 0)
                proc(c0 + 1, 1)
                issue(c0 + 4, 1)
                proc(c0 + 2, 2)
                return c2

            nt = (BCH1 - 4) // 3
            lax.fori_loop(0, nt, triple, 0)
            c = 3 * nt
            issue(c + 2, (c + 2) % 3)
            proc(c, c % 3)
            issue(c + 3, (c + 3) % 3)
            proc(c + 1, (c + 1) % 3)
            proc(c + 2, (c + 2) % 3)
            proc(c + 3, (c + 3) % 3)
            return carry

        lax.fori_loop(0, NB1, batch, 0)
        plsc.subcore_barrier()
        pltpu.sync_copy(acc.at[pl.ds(r0, RPT)],
                        out_hbm.at[cid, pl.ds(r0, RPT)])

    return _scat


_scatter1_p0 = _make_scatter1(0, 0)
_scatter1_p1 = _make_scatter1(1, 2)


_BD = 632


def _elu(x):
    return jnp.where(x > 0, x, jnp.exp(jnp.minimum(x, 0.0)) - 1.0)


def _mid_body(acc0_ref, acc1_ref, acc2_ref, acc3_ref, sv_ref, wso_ref,
              wto_ref, a2s_ref, a2t_ref, hs2_ref, ht2_ref, eset2_ref):
    s = jnp.sum(sv_ref[...], axis=0)

    def build(refs, scol):
        cols = []
        for idx, rr in enumerate(refs):
            a = rr[0] + rr[1]
            s0 = s[:, scol + 2 * idx:scol + 2 * idx + 1]
            s1 = s[:, scol + 2 * idx + 1:scol + 2 * idx + 2]
            cols.append(a[:, 0:DOUT] / (s0 + 1e-16))
            cols.append(a[:, DOUT:PAY] / (s1 + 1e-16))
        return _elu(jnp.concatenate(cols, axis=1))

    hts = build((acc0_ref, acc1_ref), 0)
    hst = build((acc2_ref, acc3_ref), 4)
    hs2 = jnp.dot(hts, wso_ref[...], preferred_element_type=jnp.float32)
    ht2 = jnp.dot(hst, wto_ref[...], preferred_element_type=jnp.float32)
    zpad = jnp.zeros((hs2.shape[0], PAY - NCLS), jnp.float32)
    hs2_ref[...] = jnp.concatenate([hs2, zpad], axis=1)
    ht2_ref[...] = jnp.concatenate([ht2, zpad], axis=1)
    eset2_ref[...] = (
        jnp.dot(hs2, a2s_ref[...], preferred_element_type=jnp.float32)
        + jnp.dot(ht2, a2t_ref[...], preferred_element_type=jnp.float32)
    )


_mid = pl.pallas_call(
    _mid_body,
    grid=(NPAD // _BD,),
    in_specs=[
        pl.BlockSpec((NC, _BD, PAY), lambda nb: (0, nb, 0)),
        pl.BlockSpec((NC, _BD, PAY), lambda nb: (0, nb, 0)),
        pl.BlockSpec((NC, _BD, PAY), lambda nb: (0, nb, 0)),
        pl.BlockSpec((NC, _BD, PAY), lambda nb: (0, nb, 0)),
        pl.BlockSpec((NW, _BD, 8), lambda nb: (0, nb, 0)),
        pl.BlockSpec((NHEADS * DOUT, NCLS), lambda nb: (0, 0)),
        pl.BlockSpec((NHEADS * DOUT, NCLS), lambda nb: (0, 0)),
        pl.BlockSpec((NCLS, 8), lambda nb: (0, 0)),
        pl.BlockSpec((NCLS, 8), lambda nb: (0, 0)),
    ],
    out_specs=[
        pl.BlockSpec((_BD, PAY), lambda nb: (nb, 0)),
        pl.BlockSpec((_BD, PAY), lambda nb: (nb, 0)),
        pl.BlockSpec((_BD, 8), lambda nb: (nb, 0)),
    ],
    out_shape=[
        jax.ShapeDtypeStruct((NPAD, PAY), jnp.float32),
        jax.ShapeDtypeStruct((NPAD, PAY), jnp.float32),
        jax.ShapeDtypeStruct((NPAD, 8), jnp.float32),
    ],
)


@functools.partial(
    pl.kernel,
    out_type=jax.ShapeDtypeStruct((2, NC, NPAD, PAY), jnp.float32),
    mesh=_MESH,
    compiler_params=_SC_PARAMS,
    scratch_types=[
        pltpu.VMEM_SHARED((NPAD, PAY), jnp.float32),
        pltpu.VMEM((NPAD * 2,), jnp.float32),
        pltpu.VMEM((BCH2 * CH,), jnp.int32),
        pltpu.VMEM((BCH2 * CH,), jnp.int32),
        pltpu.VMEM((CH,), jnp.int32),
        pltpu.VMEM((CH,), jnp.int32),
        pltpu.VMEM((CH,), jnp.int32),
        pltpu.VMEM((CH,), jnp.float32),
        pltpu.VMEM((CH, PAY), jnp.float32),
        pltpu.VMEM((CH, PAY), jnp.float32),
        pltpu.SemaphoreType.DMA,
        pltpu.SemaphoreType.DMA,
    ],
)
def _scatter2(ei_hbm, ej_hbm, hs2_hbm, ht2_hbm, eset2_hbm, out_hbm,
              acc, eset_v, kbig, sbig, kb, sb0, sb1, wbuf,
              rows0, rows1, sem0, sem1):
    cid = lax.axis_index("c")
    sid = lax.axis_index("s")
    wid = sid * NC + cid
    pltpu.sync_copy(eset2_hbm, eset_v)
    z16 = jnp.zeros((16,), jnp.float32)
    r0 = sid * RPT
    lane = lax.iota(jnp.int32, 16)

    def zero_acc():
        def zrow(r, carry):
            for cc in range(PAY // 16):
                rows0[r, pl.ds(cc * 16, 16)] = z16
            return carry

        lax.fori_loop(0, CH, zrow, 0)
        for off, nr in _ZCHUNKS:
            pltpu.sync_copy(rows0.at[pl.ds(0, nr)],
                            acc.at[pl.ds(r0 + off, nr)])

    def scale(rows):
        def body(gg, c2):
            wv = wbuf[pl.ds(gg * 16, 16)]
            for kp in range(16):
                k = gg * 16 + kp
                w = wv[kp]
                rows[k, pl.ds(0, 16)] = rows[k, pl.ds(0, 16)] * w
                rows[k, pl.ds(16, 16)] = jnp.where(lane == 0, w, 0.0)
            return c2

        lax.fori_loop(0, NG16, body, 0)

    for p, (keys_hbm, sidx_hbm, src_hbm, es_on_sidx) in enumerate((
            (ei_hbm, ej_hbm, hs2_hbm, True),
            (ej_hbm, ei_hbm, ht2_hbm, False))):

        def weights(coff):
            for kk in range(NG16):
                kk8 = kbig[pl.ds(coff + kk * 16, 16)] * 2
                ss8 = sbig[pl.ds(coff + kk * 16, 16)] * 2
                if es_on_sidx:
                    es = plsc.load_gather(eset_v, [ss8])
                    et = plsc.load_gather(
                        eset_v, [kk8 + jnp.ones((16,), jnp.int32)])
                else:
                    es = plsc.load_gather(eset_v, [kk8])
                    et = plsc.load_gather(
                        eset_v, [ss8 + jnp.ones((16,), jnp.int32)])
                x = es + et
                x = jnp.where(x >= 0, x, SLOPE * x)
                wbuf[pl.ds(kk * 16, 16)] = jnp.exp(x)

        zero_acc()
        plsc.subcore_barrier()

        def batch(b, carry):
            ib = wid * EPW + b * BCH2 * CH
            pltpu.sync_copy(keys_hbm.at[pl.ds(ib, BCH2 * CH)], kbig)
            pltpu.sync_copy(sidx_hbm.at[pl.ds(ib, BCH2 * CH)], sbig)

            _fill16(sb0, sbig, 0)
            pltpu.async_copy(src_hbm.at[sb0], rows0, sem0)

            def pair(t, c2):
                c0 = 2 * t
                _fill16(sb1, sbig, (c0 + 1) * CH)
                pltpu.async_copy(src_hbm.at[sb1], rows1, sem1)
                _fill16(kb, kbig, c0 * CH)
                weights(c0 * CH)
                pltpu.make_async_copy(src_hbm.at[sb0], rows0, sem0).wait()
                scale(rows0)
                pltpu.sync_copy(rows0, acc.at[kb], add=True)
                _fill16(sb0, sbig, (c0 + 2) * CH)
                pltpu.async_copy(src_hbm.at[sb0], rows0, sem0)
                _fill16(kb, kbig, (c0 + 1) * CH)
                weights((c0 + 1) * CH)
                pltpu.make_async_copy(src_hbm.at[sb1], rows1, sem1).wait()
                scale(rows1)
                pltpu.sync_copy(rows1, acc.at[kb], add=True)
                return c2

            lax.fori_loop(0, (BCH2 - 1) // 2, pair, 0)
            _fill16(kb, kbig, (BCH2 - 1) * CH)
            weights((BCH2 - 1) * CH)
            pltpu.make_async_copy(src_hbm.at[sb0], rows0, sem0).wait()
            scale(rows0)
            pltpu.sync_copy(rows0, acc.at[kb], add=True)
            return carry

        lax.fori_loop(0, NB2, batch, 0)
        plsc.subcore_barrier()
        pltpu.sync_copy(acc.at[pl.ds(r0, RPT)],
                        out_hbm.at[p, cid, pl.ds(r0, RPT)])


_BG = 632


def _fin_body(a2_ref, ost_ref, ots_ref):
    def fin(p0):
        a = a2_ref[p0] + a2_ref[p0 + 1]
        s = a[:, NCLS:NCLS + 1]
        v = _elu(a[:, 0:NCLS] / (s + 1e-16))
        m = jnp.max(v, axis=1, keepdims=True)
        lse = jnp.log(jnp.sum(jnp.exp(v - m), axis=1, keepdims=True)) + m
        return v - lse

    ost_ref[...] = fin(2)
    ots_ref[...] = fin(0)


_fin = pl.pallas_call(
    _fin_body,
    grid=(NPAD // _BG,),
    in_specs=[
        pl.BlockSpec((2 * NC, _BG, PAY), lambda nb: (0, nb, 0)),
    ],
    out_specs=[
        pl.BlockSpec((_BG, NCLS), lambda nb: (nb, 0)),
        pl.BlockSpec((_BG, NCLS), lambda nb: (nb, 0)),
    ],
    out_shape=[
        jax.ShapeDtypeStruct((NPAD, NCLS), jnp.float32),
        jax.ShapeDtypeStruct((NPAD, NCLS), jnp.float32),
    ],
)


@jax.jit
def kernel(input1, input2, edge_index, Ws_heads, Wt_heads, a_heads,
           Ws_out, Wt_out, a_out):
    edge_index = edge_index.astype(jnp.int32)
    ei = edge_index[0]
    ej = edge_index[1]

    pad = ((0, NPAD - N_NODES), (0, 0))
    x1p = jnp.pad(input1, pad)
    x2p = jnp.pad(input2, pad)

    ws2 = jnp.stack([
        jnp.concatenate([Ws_heads[0], Ws_heads[1]], axis=1),
        jnp.concatenate([Ws_heads[2], Ws_heads[3]], axis=1),
    ])
    wt2 = jnp.stack([
        jnp.concatenate([Wt_heads[0], Wt_heads[1]], axis=1),
        jnp.concatenate([Wt_heads[2], Wt_heads[3]], axis=1),
    ])
    as_vec = jnp.einsum("hdk,hk->dh", Ws_heads, a_heads[:, :DOUT])
    at_vec = jnp.einsum("hdk,hk->dh", Wt_heads, a_heads[:, DOUT:])
    aux_s = jnp.zeros((DIN, 8), jnp.float32).at[:, :NHEADS].set(as_vec)
    aux_t = jnp.zeros((DIN, 8), jnp.float32).at[:, NHEADS:].set(at_vec)
    a2s = jnp.zeros((NCLS, 8), jnp.float32).at[:, 0].set(a_out[:NCLS])
    a2t = jnp.zeros((NCLS, 8), jnp.float32).at[:, 1].set(a_out[NCLS:])

    hs2d, ht2d, eset = _mm1(x1p, x2p, ws2, wt2, aux_s, aux_t)

    w4 = _edge_logits(ei, ej, eset.reshape(-1))
    sv = _sums(ei, ej, w4).reshape(NW, NPAD, 8)

    a_ts0 = _scatter1_p0(ei, ej, w4, hs2d)
    a_ts1 = _scatter1_p1(ei, ej, w4, hs2d)
    a_st0 = _scatter1_p0(ej, ei, w4, ht2d)
    a_st1 = _scatter1_p1(ej, ei, w4, ht2d)

    hs2, ht2, eset2 = _mid(a_ts0, a_ts1, a_st0, a_st1, sv,
                           Ws_out, Wt_out, a2s, a2t)

    eset2c = eset2[:, :2].reshape(-1)
    a2 = _scatter2(ei, ej, hs2, ht2, eset2c)

    out_st, out_ts = _fin(a2.reshape(2 * NC, NPAD, PAY))
    return out_st[:N_NODES], out_ts[:N_NODES]
